# Initial kernel scaffold; baseline (speedup 1.0000x reference)
import jax
import jax.numpy as jnp
from jax.experimental import pallas as pl


def kernel(x, edge_index, W1, b1, W2, b2, W3, b3):
    raise NotImplementedError("write your pallas kernel here")


# speedup vs baseline: 24.9493x; 24.9493x over previous
import functools

import jax
import jax.numpy as jnp
from jax import lax
from jax.experimental import pallas as pl
from jax.experimental.pallas import tpu as pltpu
from jax.experimental.pallas import tpu_sc as plsc

NSC = 2
NTILE = 16
WIN = 80


def _make_deg_kernel(n2, nwin):
    rows_t = n2 // NTILE
    mesh = plsc.VectorSubcoreMesh(core_axis_name="c", subcore_axis_name="s")

    @functools.partial(
        pl.kernel,
        out_type=jax.ShapeDtypeStruct((NSC, n2), jnp.float32),
        mesh=mesh,
        scratch_types=[
            pltpu.VMEM((nwin, WIN), jnp.int32),
            pltpu.VMEM((rows_t,), jnp.float32),
            pltpu.VMEM((WIN,), jnp.float32),
            pltpu.VMEM_SHARED((n2,), jnp.float32),
        ],
    )
    def deg_kernel(dst_hbm, out_hbm, idx_v, zero_v, ones_v, cnt_sh):
        c = lax.axis_index("c")
        s = lax.axis_index("s")
        w = c * NTILE + s

        def zbody(i, _):
            zero_v[pl.ds(i * 16, 16)] = jnp.zeros((16,), jnp.float32)
            return 0

        lax.fori_loop(0, rows_t // 16, zbody, 0)

        def obody(i, _):
            ones_v[pl.ds(i * 16, 16)] = jnp.ones((16,), jnp.float32)
            return 0

        lax.fori_loop(0, WIN // 16, obody, 0)

        pltpu.sync_copy(dst_hbm.at[w], idx_v)
        pltpu.sync_copy(zero_v, cnt_sh.at[pl.ds(s * rows_t, rows_t)])
        plsc.subcore_barrier()

        def body(j, _):
            pltpu.sync_copy(ones_v, cnt_sh.at[idx_v.at[j]], add=True)
            return 0

        lax.fori_loop(0, nwin, body, 0)
        plsc.subcore_barrier()
        pltpu.sync_copy(cnt_sh.at[pl.ds(s * rows_t, rows_t)],
                        out_hbm.at[c, pl.ds(s * rows_t, rows_t)])

    return deg_kernel


def _make_agg_kernel(n2, nwin):
    rows_t = n2 // NTILE
    mesh = plsc.VectorSubcoreMesh(core_axis_name="c", subcore_axis_name="s")

    @functools.partial(
        pl.kernel,
        out_type=jax.ShapeDtypeStruct((NSC, n2, 128), jnp.float32),
        mesh=mesh,
        scratch_types=[
            pltpu.VMEM((nwin * WIN,), jnp.int32),
            pltpu.VMEM((nwin, WIN), jnp.int32),
            pltpu.VMEM((WIN, 128), jnp.float32),
            pltpu.VMEM((WIN, 128), jnp.float32),
            pltpu.SemaphoreType.DMA,
            pltpu.SemaphoreType.DMA,
            pltpu.VMEM_SHARED((n2, 128), jnp.float32),
        ],
    )
    def agg_kernel(g_hbm, src_hbm, dst_hbm, out_hbm,
                   src_v, dst_v, r0, r1, m0, m1, s_sh):
        c = lax.axis_index("c")
        s = lax.axis_index("s")
        row0 = s * rows_t

        pltpu.sync_copy(src_hbm.at[c, s], src_v)
        pltpu.sync_copy(dst_hbm.at[c, s], dst_v)

        @pl.when(c == 0)
        def _():
            pltpu.sync_copy(g_hbm.at[pl.ds(row0, rows_t)],
                            s_sh.at[pl.ds(row0, rows_t)])

        @pl.when(c == 1)
        def _():
            def zb(i, _):
                r0[i // 8, pl.ds((i % 8) * 16, 16)] = jnp.zeros((16,),
                                                               jnp.float32)
                return 0

            lax.fori_loop(0, WIN * 8, zb, 0)
            for m in range(rows_t // WIN):
                pltpu.sync_copy(r0, s_sh.at[pl.ds(row0 + m * WIN, WIN)])

        plsc.subcore_barrier()

        def body(i, _):
            j = 2 * i
            cp0 = pltpu.async_copy(
                g_hbm.at[src_v.at[pl.ds(j * WIN, WIN)]], r0, m0)
            cp1 = pltpu.async_copy(
                g_hbm.at[src_v.at[pl.ds((j + 1) * WIN, WIN)]], r1, m1)
            cp0.wait()
            pltpu.sync_copy(r0, s_sh.at[dst_v.at[j]], add=True)
            cp1.wait()
            pltpu.sync_copy(r1, s_sh.at[dst_v.at[j + 1]], add=True)
            return 0

        lax.fori_loop(0, nwin // 2, body, 0)
        if nwin % 2:
            jlast = nwin - 1
            pltpu.async_copy(
                g_hbm.at[src_v.at[pl.ds(jlast * WIN, WIN)]], r0, m0).wait()
            pltpu.sync_copy(r0, s_sh.at[dst_v.at[jlast]], add=True)

        plsc.subcore_barrier()
        pltpu.sync_copy(s_sh.at[pl.ds(row0, rows_t)],
                        out_hbm.at[c, pl.ds(row0, rows_t)])

    return agg_kernel


def _first_layer(x, w, d01, n2, nrb, rb):
    def body(x_ref, w_ref, d_ref, o_ref):
        dinv = lax.rsqrt(d_ref[:, 0:1] + d_ref[:, 1:2] + 1.0)
        o_ref[...] = (
            jnp.dot(x_ref[...], w_ref[...], preferred_element_type=jnp.float32)
            * dinv
        )

    return pl.pallas_call(
        body,
        grid=(nrb,),
        in_specs=[
            pl.BlockSpec((rb, 128), lambda i: (i, 0)),
            pl.BlockSpec((128, 128), lambda i: (0, 0)),
            pl.BlockSpec((rb, 2), lambda i: (i, 0)),
        ],
        out_specs=pl.BlockSpec((rb, 128), lambda i: (i, 0)),
        out_shape=jax.ShapeDtypeStruct((n2, 128), jnp.float32),
    )(x, w, d01)


def _mid_layer(s1, b, w, d01, n2, nrb, rb):
    def body(sa_ref, sb_ref, d_ref, b_ref, w_ref, o_ref):
        dinv = lax.rsqrt(d_ref[:, 0:1] + d_ref[:, 1:2] + 1.0)
        h = (sa_ref[0] + sb_ref[0]) * dinv + b_ref[...]
        h = jnp.maximum(h, 0.0)
        o_ref[...] = (
            jnp.dot(h, w_ref[...], preferred_element_type=jnp.float32) * dinv
        )

    return pl.pallas_call(
        body,
        grid=(nrb,),
        in_specs=[
            pl.BlockSpec((1, rb, 128), lambda i: (0, i, 0)),
            pl.BlockSpec((1, rb, 128), lambda i: (1, i, 0)),
            pl.BlockSpec((rb, 2), lambda i: (i, 0)),
            pl.BlockSpec((1, 128), lambda i: (0, 0)),
            pl.BlockSpec((128, 128), lambda i: (0, 0)),
        ],
        out_specs=pl.BlockSpec((rb, 128), lambda i: (i, 0)),
        out_shape=jax.ShapeDtypeStruct((n2, 128), jnp.float32),
    )(s1, s1, d01, b.reshape(1, 128), w)


def _final_layer(s2, b, w3p, b3p, d01, n, nrb, rb):
    inv_n = 1.0 / n

    def body(sa_ref, sb_ref, d_ref, b_ref, w_ref, b3_ref, o_ref, acc_ref):
        i = pl.program_id(0)
        dinv = lax.rsqrt(d_ref[:, 0:1] + d_ref[:, 1:2] + 1.0)
        h = (sa_ref[0] + sb_ref[0]) * dinv + b_ref[...]
        h = jnp.maximum(h, 0.0)
        colsum = jnp.sum(h, axis=0, keepdims=True)

        @pl.when(i == 0)
        def _():
            acc_ref[...] = jnp.zeros_like(acc_ref)

        acc_ref[0:1, :] = acc_ref[0:1, :] + colsum

        @pl.when(i == nrb - 1)
        def _():
            pooled = acc_ref[0:1, :] * inv_n
            o_ref[...] = (
                jnp.dot(pooled, w_ref[...], preferred_element_type=jnp.float32)
                + b3_ref[...]
            )

    return pl.pallas_call(
        body,
        grid=(nrb,),
        in_specs=[
            pl.BlockSpec((1, rb, 128), lambda i: (0, i, 0)),
            pl.BlockSpec((1, rb, 128), lambda i: (1, i, 0)),
            pl.BlockSpec((rb, 2), lambda i: (i, 0)),
            pl.BlockSpec((1, 128), lambda i: (0, 0)),
            pl.BlockSpec((128, 128), lambda i: (0, 0)),
            pl.BlockSpec((1, 128), lambda i: (0, 0)),
        ],
        out_specs=pl.BlockSpec((1, 128), lambda i: (0, 0)),
        out_shape=jax.ShapeDtypeStruct((1, 128), jnp.float32),
        scratch_shapes=[pltpu.VMEM((8, 128), jnp.float32)],
    )(s2, s2, d01, b.reshape(1, 128), w3p, b3p)


def kernel(x, edge_index, W1, b1, W2, b2, W3, b3):
    n, d_in = x.shape
    e = edge_index.shape[1]
    c_out = W3.shape[1]
    assert d_in == 128 and W1.shape[1] == 128 and W2.shape[1] == 128
    assert e % (NSC * NTILE * WIN) == 0 and n % NTILE == 0

    src = edge_index[0]
    dst = edge_index[1]

    n2 = ((n + NTILE * 16 - 1) // (NTILE * 16)) * NTILE * 16
    ew = e // (NSC * NTILE)
    nwin = ew // WIN
    dst_deg = dst.reshape(NSC * NTILE, nwin, WIN)
    src_agg = src.reshape(NSC, NTILE, ew)
    dst_agg = dst.reshape(NSC, NTILE, nwin, WIN)

    w3p = jnp.pad(W3, ((0, 0), (0, 128 - c_out)))
    b3p = jnp.pad(b3, (0, 128 - c_out)).reshape(1, 128)

    deg_parts = _make_deg_kernel(n2, nwin)(dst_deg)
    d01 = deg_parts[:, :n].T

    rb = 2000 if n % 2000 == 0 else 8 * (n // 8)
    nrb = n // rb

    agg = _make_agg_kernel(n2, nwin)

    g1 = _first_layer(x, W1, d01, n2, nrb, rb)
    s1 = agg(g1, src_agg, dst_agg)
    g2 = _mid_layer(s1, b1, W2, d01, n2, nrb, rb)
    s2 = agg(g2, src_agg, dst_agg)
    out = _final_layer(s2, b2, w3p, b3p, d01, n, nrb, rb)
    return out[:, :c_out]


# speedup vs baseline: 25.7150x; 1.0307x over previous
import functools

import jax
import jax.numpy as jnp
from jax import lax
from jax.experimental import pallas as pl
from jax.experimental.pallas import tpu as pltpu
from jax.experimental.pallas import tpu_sc as plsc

NSC = 2
NTILE = 16
WIN = 80


def _make_deg_kernel(n2, nwin):
    rows_t = n2 // NTILE
    mesh = plsc.VectorSubcoreMesh(core_axis_name="c", subcore_axis_name="s")

    @functools.partial(
        pl.kernel,
        out_type=jax.ShapeDtypeStruct((NSC, n2), jnp.float32),
        mesh=mesh,
        scratch_types=[
            pltpu.VMEM((nwin, WIN), jnp.int32),
            pltpu.VMEM((rows_t,), jnp.float32),
            pltpu.VMEM((WIN,), jnp.float32),
            pltpu.VMEM_SHARED((n2,), jnp.float32),
        ],
    )
    def deg_kernel(dst_hbm, out_hbm, idx_v, zero_v, ones_v, cnt_sh):
        c = lax.axis_index("c")
        s = lax.axis_index("s")
        w = c * NTILE + s

        def zbody(i, _):
            zero_v[pl.ds(i * 16, 16)] = jnp.zeros((16,), jnp.float32)
            return 0

        lax.fori_loop(0, rows_t // 16, zbody, 0)

        def obody(i, _):
            ones_v[pl.ds(i * 16, 16)] = jnp.ones((16,), jnp.float32)
            return 0

        lax.fori_loop(0, WIN // 16, obody, 0)

        pltpu.sync_copy(dst_hbm.at[w], idx_v)
        pltpu.sync_copy(zero_v, cnt_sh.at[pl.ds(s * rows_t, rows_t)])
        plsc.subcore_barrier()

        def body(j, _):
            pltpu.sync_copy(ones_v, cnt_sh.at[idx_v.at[j]], add=True)
            return 0

        lax.fori_loop(0, nwin, body, 0)
        plsc.subcore_barrier()
        pltpu.sync_copy(cnt_sh.at[pl.ds(s * rows_t, rows_t)],
                        out_hbm.at[c, pl.ds(s * rows_t, rows_t)])

    return deg_kernel


def _make_agg_kernel(n2, nwin):
    rows_t = n2 // NTILE
    mesh = plsc.VectorSubcoreMesh(core_axis_name="c", subcore_axis_name="s")

    @functools.partial(
        pl.kernel,
        out_type=jax.ShapeDtypeStruct((NSC, n2, 128), jnp.float32),
        mesh=mesh,
        scratch_types=[
            pltpu.VMEM((nwin * WIN,), jnp.int32),
            pltpu.VMEM((nwin, WIN), jnp.int32),
            pltpu.VMEM((WIN, 128), jnp.float32),
            pltpu.VMEM((WIN, 128), jnp.float32),
            pltpu.SemaphoreType.DMA,
            pltpu.SemaphoreType.DMA,
            pltpu.SemaphoreType.DMA,
            pltpu.SemaphoreType.DMA,
            pltpu.VMEM_SHARED((n2, 128), jnp.float32),
        ],
    )
    def agg_kernel(g_hbm, src_hbm, dst_hbm, out_hbm,
                   src_v, dst_v, r0, r1, m0, m1, c0, c1, s_sh):
        c = lax.axis_index("c")
        s = lax.axis_index("s")
        row0 = s * rows_t

        pltpu.sync_copy(src_hbm.at[c, s], src_v)
        pltpu.sync_copy(dst_hbm.at[c, s], dst_v)

        @pl.when(c == 0)
        def _():
            pltpu.sync_copy(g_hbm.at[pl.ds(row0, rows_t)],
                            s_sh.at[pl.ds(row0, rows_t)])

        @pl.when(c == 1)
        def _():
            def zb(i, _):
                r0[i // 8, pl.ds((i % 8) * 16, 16)] = jnp.zeros((16,),
                                                               jnp.float32)
                return 0

            lax.fori_loop(0, WIN * 8, zb, 0)
            for m in range(rows_t // WIN):
                pltpu.sync_copy(r0, s_sh.at[pl.ds(row0 + m * WIN, WIN)])

        plsc.subcore_barrier()

        npair = nwin // 2

        def gather(j, buf, sem):
            return pltpu.async_copy(
                g_hbm.at[src_v.at[pl.ds(j * WIN, WIN)]], buf, sem)

        def gwait(j, buf, sem):
            pltpu.make_async_copy(
                g_hbm.at[src_v.at[pl.ds(j * WIN, WIN)]], buf, sem).wait()

        gather(0, r0, m0)
        gather(1, r1, m1)

        def body(i, _):
            j = 2 * i
            gwait(j, r0, m0)
            sc0 = pltpu.async_copy(r0, s_sh.at[dst_v.at[j]], c0, add=True)
            gwait(j + 1, r1, m1)
            sc1 = pltpu.async_copy(r1, s_sh.at[dst_v.at[j + 1]], c1, add=True)
            sc0.wait()

            @pl.when(i < npair - 1)
            def _():
                gather(j + 2, r0, m0)

            sc1.wait()

            @pl.when(i < npair - 1)
            def _():
                gather(j + 3, r1, m1)

            return 0

        lax.fori_loop(0, npair, body, 0)
        if nwin % 2:
            jlast = nwin - 1
            gather(jlast, r0, m0).wait()
            pltpu.sync_copy(r0, s_sh.at[dst_v.at[jlast]], add=True)

        plsc.subcore_barrier()
        pltpu.sync_copy(s_sh.at[pl.ds(row0, rows_t)],
                        out_hbm.at[c, pl.ds(row0, rows_t)])

    return agg_kernel


def _first_layer(x, w, d01, n2, nrb, rb):
    def body(x_ref, w_ref, d_ref, o_ref):
        dinv = lax.rsqrt(d_ref[:, 0:1] + d_ref[:, 1:2] + 1.0)
        o_ref[...] = (
            jnp.dot(x_ref[...], w_ref[...], preferred_element_type=jnp.float32)
            * dinv
        )

    return pl.pallas_call(
        body,
        grid=(nrb,),
        in_specs=[
            pl.BlockSpec((rb, 128), lambda i: (i, 0)),
            pl.BlockSpec((128, 128), lambda i: (0, 0)),
            pl.BlockSpec((rb, 2), lambda i: (i, 0)),
        ],
        out_specs=pl.BlockSpec((rb, 128), lambda i: (i, 0)),
        out_shape=jax.ShapeDtypeStruct((n2, 128), jnp.float32),
    )(x, w, d01)


def _mid_layer(s1, b, w, d01, n2, nrb, rb):
    def body(sa_ref, sb_ref, d_ref, b_ref, w_ref, o_ref):
        dinv = lax.rsqrt(d_ref[:, 0:1] + d_ref[:, 1:2] + 1.0)
        h = (sa_ref[0] + sb_ref[0]) * dinv + b_ref[...]
        h = jnp.maximum(h, 0.0)
        o_ref[...] = (
            jnp.dot(h, w_ref[...], preferred_element_type=jnp.float32) * dinv
        )

    return pl.pallas_call(
        body,
        grid=(nrb,),
        in_specs=[
            pl.BlockSpec((1, rb, 128), lambda i: (0, i, 0)),
            pl.BlockSpec((1, rb, 128), lambda i: (1, i, 0)),
            pl.BlockSpec((rb, 2), lambda i: (i, 0)),
            pl.BlockSpec((1, 128), lambda i: (0, 0)),
            pl.BlockSpec((128, 128), lambda i: (0, 0)),
        ],
        out_specs=pl.BlockSpec((rb, 128), lambda i: (i, 0)),
        out_shape=jax.ShapeDtypeStruct((n2, 128), jnp.float32),
    )(s1, s1, d01, b.reshape(1, 128), w)


def _final_layer(s2, b, w3p, b3p, d01, n, nrb, rb):
    inv_n = 1.0 / n

    def body(sa_ref, sb_ref, d_ref, b_ref, w_ref, b3_ref, o_ref, acc_ref):
        i = pl.program_id(0)
        dinv = lax.rsqrt(d_ref[:, 0:1] + d_ref[:, 1:2] + 1.0)
        h = (sa_ref[0] + sb_ref[0]) * dinv + b_ref[...]
        h = jnp.maximum(h, 0.0)
        colsum = jnp.sum(h, axis=0, keepdims=True)

        @pl.when(i == 0)
        def _():
            acc_ref[...] = jnp.zeros_like(acc_ref)

        acc_ref[0:1, :] = acc_ref[0:1, :] + colsum

        @pl.when(i == nrb - 1)
        def _():
            pooled = acc_ref[0:1, :] * inv_n
            o_ref[...] = (
                jnp.dot(pooled, w_ref[...], preferred_element_type=jnp.float32)
                + b3_ref[...]
            )

    return pl.pallas_call(
        body,
        grid=(nrb,),
        in_specs=[
            pl.BlockSpec((1, rb, 128), lambda i: (0, i, 0)),
            pl.BlockSpec((1, rb, 128), lambda i: (1, i, 0)),
            pl.BlockSpec((rb, 2), lambda i: (i, 0)),
            pl.BlockSpec((1, 128), lambda i: (0, 0)),
            pl.BlockSpec((128, 128), lambda i: (0, 0)),
            pl.BlockSpec((1, 128), lambda i: (0, 0)),
        ],
        out_specs=pl.BlockSpec((1, 128), lambda i: (0, 0)),
        out_shape=jax.ShapeDtypeStruct((1, 128), jnp.float32),
        scratch_shapes=[pltpu.VMEM((8, 128), jnp.float32)],
    )(s2, s2, d01, b.reshape(1, 128), w3p, b3p)


def kernel(x, edge_index, W1, b1, W2, b2, W3, b3):
    n, d_in = x.shape
    e = edge_index.shape[1]
    c_out = W3.shape[1]
    assert d_in == 128 and W1.shape[1] == 128 and W2.shape[1] == 128
    assert e % (NSC * NTILE * WIN) == 0 and n % NTILE == 0

    src = edge_index[0]
    dst = edge_index[1]

    n2 = ((n + NTILE * 16 - 1) // (NTILE * 16)) * NTILE * 16
    ew = e // (NSC * NTILE)
    nwin = ew // WIN
    dst_deg = dst.reshape(NSC * NTILE, nwin, WIN)
    src_agg = src.reshape(NSC, NTILE, ew)
    dst_agg = dst.reshape(NSC, NTILE, nwin, WIN)

    w3p = jnp.pad(W3, ((0, 0), (0, 128 - c_out)))
    b3p = jnp.pad(b3, (0, 128 - c_out)).reshape(1, 128)

    deg_parts = _make_deg_kernel(n2, nwin)(dst_deg)
    d01 = deg_parts[:, :n].T

    rb = 2000 if n % 2000 == 0 else 8 * (n // 8)
    nrb = n // rb

    agg = _make_agg_kernel(n2, nwin)

    g1 = _first_layer(x, W1, d01, n2, nrb, rb)
    s1 = agg(g1, src_agg, dst_agg)
    g2 = _mid_layer(s1, b1, W2, d01, n2, nrb, rb)
    s2 = agg(g2, src_agg, dst_agg)
    out = _final_layer(s2, b2, w3p, b3p, d01, n, nrb, rb)
    return out[:, :c_out]


# speedup vs baseline: 35.6064x; 1.3847x over previous
import functools

import jax
import jax.numpy as jnp
from jax import lax
from jax.experimental import pallas as pl
from jax.experimental.pallas import tpu as pltpu
from jax.experimental.pallas import tpu_sc as plsc

NSC = 2
NTILE = 16
WIN = 80


def _make_deg_kernel(n2, nwin):
    rows_t = n2 // NTILE
    mesh = plsc.VectorSubcoreMesh(core_axis_name="c", subcore_axis_name="s")

    @functools.partial(
        pl.kernel,
        out_type=jax.ShapeDtypeStruct((NSC, n2), jnp.float32),
        mesh=mesh,
        scratch_types=[
            pltpu.VMEM((nwin, WIN), jnp.int32),
            pltpu.VMEM((rows_t,), jnp.float32),
            pltpu.VMEM((WIN,), jnp.float32),
            pltpu.VMEM_SHARED((n2,), jnp.float32),
        ],
    )
    def deg_kernel(dst_hbm, out_hbm, idx_v, zero_v, ones_v, cnt_sh):
        c = lax.axis_index("c")
        s = lax.axis_index("s")
        w = c * NTILE + s

        def zbody(i, _):
            zero_v[pl.ds(i * 16, 16)] = jnp.zeros((16,), jnp.float32)
            return 0

        lax.fori_loop(0, rows_t // 16, zbody, 0)

        def obody(i, _):
            ones_v[pl.ds(i * 16, 16)] = jnp.ones((16,), jnp.float32)
            return 0

        lax.fori_loop(0, WIN // 16, obody, 0)

        pltpu.sync_copy(dst_hbm.at[w], idx_v)
        pltpu.sync_copy(zero_v, cnt_sh.at[pl.ds(s * rows_t, rows_t)])
        plsc.subcore_barrier()

        def body(j, _):
            pltpu.sync_copy(ones_v, cnt_sh.at[idx_v.at[j]], add=True)
            return 0

        lax.fori_loop(0, nwin, body, 0)
        plsc.subcore_barrier()
        pltpu.sync_copy(cnt_sh.at[pl.ds(s * rows_t, rows_t)],
                        out_hbm.at[c, pl.ds(s * rows_t, rows_t)])

    return deg_kernel


def _make_agg_kernel(n2, nwin):
    rows_t = n2 // NTILE
    mesh = plsc.VectorSubcoreMesh(core_axis_name="c", subcore_axis_name="s")

    CH = 24
    ntrip = nwin // 3

    @functools.partial(
        pl.kernel,
        out_type=jax.ShapeDtypeStruct((NSC, n2, 128), jnp.float32),
        mesh=mesh,
        scratch_types=[
            pltpu.VMEM((nwin * WIN,), jnp.int32),
            pltpu.VMEM((CH, WIN), jnp.int32),
            pltpu.VMEM((WIN, 128), jnp.float32),
            pltpu.VMEM((WIN, 128), jnp.float32),
            pltpu.VMEM((WIN, 128), jnp.float32),
            pltpu.SemaphoreType.DMA,
            pltpu.SemaphoreType.DMA,
            pltpu.SemaphoreType.DMA,
            pltpu.SemaphoreType.DMA,
            pltpu.SemaphoreType.DMA,
            pltpu.SemaphoreType.DMA,
            pltpu.VMEM_SHARED((n2, 128), jnp.float32),
        ],
    )
    def agg_kernel(g_hbm, src_hbm, dst_hbm, out_hbm, src_v, dst_v,
                   r0, r1, r2, m0, m1, m2, c0, c1, c2, s_sh):
        c = lax.axis_index("c")
        s = lax.axis_index("s")
        row0 = s * rows_t
        bufs = (r0, r1, r2)
        gsem = (m0, m1, m2)
        ssem = (c0, c1, c2)

        pltpu.sync_copy(src_hbm.at[c, s], src_v)

        @pl.when(c == 0)
        def _():
            pltpu.sync_copy(g_hbm.at[pl.ds(row0, rows_t)],
                            s_sh.at[pl.ds(row0, rows_t)])

        @pl.when(c == 1)
        def _():
            def zb(i, _):
                r0[i // 8, pl.ds((i % 8) * 16, 16)] = jnp.zeros((16,),
                                                               jnp.float32)
                return 0

            lax.fori_loop(0, WIN * 8, zb, 0)
            for m in range(rows_t // WIN):
                pltpu.sync_copy(r0, s_sh.at[pl.ds(row0 + m * WIN, WIN)])

        plsc.subcore_barrier()

        def gather(j, k):
            return pltpu.async_copy(
                g_hbm.at[src_v.at[pl.ds(j * WIN, WIN)]], bufs[k], gsem[k])

        def gwait(j, k):
            pltpu.make_async_copy(
                g_hbm.at[src_v.at[pl.ds(j * WIN, WIN)]], bufs[k],
                gsem[k]).wait()

        def swait(k):
            pltpu.make_async_copy(bufs[k], s_sh.at[dst_v.at[0]],
                                  ssem[k]).wait()

        gather(0, 0)
        gather(1, 1)

        def body(t, _):
            j0 = 3 * t
            tm = t % (CH // 3)
            for k in range(3):
                if k == 0:
                    @pl.when(t > 0)
                    def _():
                        swait(2)

                    @pl.when(tm == 0)
                    def _():
                        pltpu.sync_copy(
                            dst_hbm.at[c, s, pl.ds((t // (CH // 3)) * CH, CH)],
                            dst_v)
                else:
                    swait(k - 1)
                gwait(j0 + k, k)
                pltpu.async_copy(bufs[k], s_sh.at[dst_v.at[3 * tm + k]],
                                 ssem[k], add=True)
                gather(j0 + k + 2, (k + 2) % 3)
            return 0

        lax.fori_loop(0, ntrip, body, 0)
        base = (3 * ntrip // CH) * CH
        for jt in range(3 * ntrip, nwin):
            k = jt % 3
            swait((jt - 1) % 3)
            gwait(jt, k)
            pltpu.async_copy(bufs[k], s_sh.at[dst_v.at[jt - base]],
                             ssem[k], add=True)
        swait((nwin - 1) % 3)

        plsc.subcore_barrier()
        pltpu.sync_copy(s_sh.at[pl.ds(row0, rows_t)],
                        out_hbm.at[c, pl.ds(row0, rows_t)])

    return agg_kernel


def _first_layer(x, w, d01, n2, nrb, rb):
    def body(x_ref, w_ref, d_ref, o_ref):
        dinv = lax.rsqrt(d_ref[:, 0:1] + d_ref[:, 1:2] + 1.0)
        o_ref[...] = (
            jnp.dot(x_ref[...], w_ref[...], preferred_element_type=jnp.float32)
            * dinv
        )

    return pl.pallas_call(
        body,
        grid=(nrb,),
        in_specs=[
            pl.BlockSpec((rb, 128), lambda i: (i, 0)),
            pl.BlockSpec((128, 128), lambda i: (0, 0)),
            pl.BlockSpec((rb, 2), lambda i: (i, 0)),
        ],
        out_specs=pl.BlockSpec((rb, 128), lambda i: (i, 0)),
        out_shape=jax.ShapeDtypeStruct((n2, 128), jnp.float32),
    )(x, w, d01)


def _mid_layer(s1, b, w, d01, n2, nrb, rb):
    def body(sa_ref, sb_ref, d_ref, b_ref, w_ref, o_ref):
        dinv = lax.rsqrt(d_ref[:, 0:1] + d_ref[:, 1:2] + 1.0)
        h = (sa_ref[0] + sb_ref[0]) * dinv + b_ref[...]
        h = jnp.maximum(h, 0.0)
        o_ref[...] = (
            jnp.dot(h, w_ref[...], preferred_element_type=jnp.float32) * dinv
        )

    return pl.pallas_call(
        body,
        grid=(nrb,),
        in_specs=[
            pl.BlockSpec((1, rb, 128), lambda i: (0, i, 0)),
            pl.BlockSpec((1, rb, 128), lambda i: (1, i, 0)),
            pl.BlockSpec((rb, 2), lambda i: (i, 0)),
            pl.BlockSpec((1, 128), lambda i: (0, 0)),
            pl.BlockSpec((128, 128), lambda i: (0, 0)),
        ],
        out_specs=pl.BlockSpec((rb, 128), lambda i: (i, 0)),
        out_shape=jax.ShapeDtypeStruct((n2, 128), jnp.float32),
    )(s1, s1, d01, b.reshape(1, 128), w)


def _final_layer(s2, b, w3p, b3p, d01, n, nrb, rb):
    inv_n = 1.0 / n

    def body(sa_ref, sb_ref, d_ref, b_ref, w_ref, b3_ref, o_ref, acc_ref):
        i = pl.program_id(0)
        dinv = lax.rsqrt(d_ref[:, 0:1] + d_ref[:, 1:2] + 1.0)
        h = (sa_ref[0] + sb_ref[0]) * dinv + b_ref[...]
        h = jnp.maximum(h, 0.0)
        colsum = jnp.sum(h, axis=0, keepdims=True)

        @pl.when(i == 0)
        def _():
            acc_ref[...] = jnp.zeros_like(acc_ref)

        acc_ref[0:1, :] = acc_ref[0:1, :] + colsum

        @pl.when(i == nrb - 1)
        def _():
            pooled = acc_ref[0:1, :] * inv_n
            o_ref[...] = (
                jnp.dot(pooled, w_ref[...], preferred_element_type=jnp.float32)
                + b3_ref[...]
            )

    return pl.pallas_call(
        body,
        grid=(nrb,),
        in_specs=[
            pl.BlockSpec((1, rb, 128), lambda i: (0, i, 0)),
            pl.BlockSpec((1, rb, 128), lambda i: (1, i, 0)),
            pl.BlockSpec((rb, 2), lambda i: (i, 0)),
            pl.BlockSpec((1, 128), lambda i: (0, 0)),
            pl.BlockSpec((128, 128), lambda i: (0, 0)),
            pl.BlockSpec((1, 128), lambda i: (0, 0)),
        ],
        out_specs=pl.BlockSpec((1, 128), lambda i: (0, 0)),
        out_shape=jax.ShapeDtypeStruct((1, 128), jnp.float32),
        scratch_shapes=[pltpu.VMEM((8, 128), jnp.float32)],
    )(s2, s2, d01, b.reshape(1, 128), w3p, b3p)


def kernel(x, edge_index, W1, b1, W2, b2, W3, b3):
    n, d_in = x.shape
    e = edge_index.shape[1]
    c_out = W3.shape[1]
    assert d_in == 128 and W1.shape[1] == 128 and W2.shape[1] == 128
    assert e % (NSC * NTILE * WIN) == 0 and n % NTILE == 0

    src = edge_index[0]
    dst = edge_index[1]

    n2 = ((n + NTILE * 16 - 1) // (NTILE * 16)) * NTILE * 16
    ew = e // (NSC * NTILE)
    nwin = ew // WIN
    dst_deg = dst.reshape(NSC * NTILE, nwin, WIN)
    src_agg = src.reshape(NSC, NTILE, ew)
    nwin_pad = ((nwin + 23) // 24) * 24
    dst_agg = jnp.pad(dst.reshape(NSC, NTILE, nwin, WIN),
                      ((0, 0), (0, 0), (0, nwin_pad - nwin), (0, 0)))

    w3p = jnp.pad(W3, ((0, 0), (0, 128 - c_out)))
    b3p = jnp.pad(b3, (0, 128 - c_out)).reshape(1, 128)

    deg_parts = _make_deg_kernel(n2, nwin)(dst_deg)
    d01 = deg_parts[:, :n].T

    rb = 2000 if n % 2000 == 0 else 8 * (n // 8)
    nrb = n // rb

    agg = _make_agg_kernel(n2, nwin)

    g1 = _first_layer(x, W1, d01, n2, nrb, rb)
    s1 = agg(g1, src_agg, dst_agg)
    g2 = _mid_layer(s1, b1, W2, d01, n2, nrb, rb)
    s2 = agg(g2, src_agg, dst_agg)
    out = _final_layer(s2, b2, w3p, b3p, d01, n, nrb, rb)
    return out[:, :c_out]
